# baseline (speedup 1.0000x reference)
import functools

import jax
import jax.numpy as jnp
from jax import lax
from jax.experimental import pallas as pl
from jax.experimental.pallas import tpu as pltpu
from jax.experimental.pallas import tpu_sc as plsc

B = 4096
F = 26
V = 100000
D = 16
C = 13

_NW = 32
_NG = F * 2
_CW = 5760
_VMAIN = 99968
_NCH = 18
_TAILW = V - _VMAIN


def _sc_gather_t(tbl, idx3, tail3):
    mesh = plsc.VectorSubcoreMesh(core_axis_name="c", subcore_axis_name="s")

    @functools.partial(
        pl.kernel,
        mesh=mesh,
        out_type=jax.ShapeDtypeStruct((_NG, 8, B), jnp.float32),
        scratch_types=[
            pltpu.VMEM((2, 8, _CW), jnp.float32),
            pltpu.VMEM((8, 512), jnp.int32),
            pltpu.VMEM((8, B), jnp.float32),
            pltpu.VMEM((8, _TAILW), jnp.float32),
            pltpu.SemaphoreType.DMA,
            pltpu.SemaphoreType.DMA,
        ],
        compiler_params=pltpu.CompilerParams(
            use_tc_tiling_on_sc=True, needs_layout_passes=False),
    )
    def k(tbl_hbm, idx_hbm, tail_hbm, out_hbm, slab_v, idx_v, out_v,
          tail_v, sem0, sem1):
        wid = lax.axis_index("s") * 2 + lax.axis_index("c")
        iota16 = lax.iota(jnp.int32, 16)
        zeros16 = jnp.zeros((16,), jnp.float32)

        def chunk_off(kk):
            return lax.min(kk * _CW, _VMAIN - _CW)

        def extract(slab, lo, width, f_unused):
            for s in range(8):
                def inner(l, c2):
                    vi = idx_v[s, pl.ds(l * 16, 16)]
                    rel = vi - lo
                    m = (rel >= 0) & (rel < width)
                    vloc = lax.max(lax.min(rel, width - 1), 0)
                    bpos = s * 512 + l * 16 + iota16
                    for sd in range(8):
                        vals = plsc.load_gather(
                            slab, [jnp.full((16,), sd, jnp.int32), vloc])
                        plsc.store_scatter(
                            out_v, [jnp.full((16,), sd, jnp.int32), bpos],
                            vals, mask=m)
                    return c2

                lax.fori_loop(0, 32, inner, 0)

        def do_group(gi, carry):
            g = wid + gi * _NW

            @pl.when(g < _NG)
            def _():
                f = g // 2
                dt = g % 2
                pltpu.sync_copy(idx_hbm.at[f], idx_v)
                pltpu.sync_copy(tail_hbm.at[g], tail_v)

                def zero(l, c2):
                    for s in range(8):
                        out_v[s, pl.ds(l * 16, 16)] = zeros16
                    return c2

                lax.fori_loop(0, B // 16, zero, 0)

                def fire(kk, buf):
                    pltpu.async_copy(
                        tbl_hbm.at[f, pl.ds(dt * 8, 8),
                                   pl.ds(chunk_off(kk), _CW)],
                        slab_v.at[buf],
                        sem0 if buf == 0 else sem1)

                def drain(buf):
                    pltpu.make_async_copy(
                        tbl_hbm.at[0, pl.ds(0, 8), pl.ds(0, _CW)],
                        slab_v.at[buf],
                        sem0 if buf == 0 else sem1).wait()

                fire(0, 0)

                def step(mm, c2):
                    k0 = mm * 2
                    drain(0)

                    @pl.when(k0 + 1 < _NCH)
                    def _():
                        fire(k0 + 1, 1)

                    extract(slab_v.at[0], chunk_off(k0), _CW, f)

                    @pl.when(k0 + 1 < _NCH)
                    def _():
                        drain(1)

                        @pl.when(k0 + 2 < _NCH)
                        def _():
                            fire(k0 + 2, 0)

                        extract(slab_v.at[1], chunk_off(k0 + 1), _CW, f)

                    return c2

                lax.fori_loop(0, (_NCH + 1) // 2, step, 0)
                extract(tail_v, _VMAIN, _TAILW, f)
                pltpu.sync_copy(out_v, out_hbm.at[g])

            return carry

        lax.fori_loop(0, 2, do_group, 0)

    return k(tbl, idx3, tail3)


def _mlp(embt, xc, w1e, w1c, b1, g1, be1, w2, b2, g2, be2, w3, b3, gc, bc):
    def body(embt_ref, xc_ref, w1e_ref, w1c_ref, b1_ref, g1_ref, be1_ref,
             w2_ref, b2_ref, g2_ref, be2_ref, w3_ref, b3_ref, gc_ref,
             bc_ref, out_ref):
        hp = jax.lax.Precision.HIGHEST
        x = xc_ref[...]
        m = jnp.mean(x, axis=0, keepdims=True)
        v = jnp.mean((x - m) * (x - m), axis=0, keepdims=True)
        xn = (x - m) * lax.rsqrt(v + 1e-5) * gc_ref[...] + bc_ref[...]

        h = jax.lax.dot_general(
            embt_ref[...], w1e_ref[...],
            dimension_numbers=(((0,), (0,)), ((), ())),
            preferred_element_type=jnp.float32, precision=hp)
        h = h + jnp.dot(xn, w1c_ref[...],
                        preferred_element_type=jnp.float32, precision=hp)
        h = jnp.maximum(h + b1_ref[...], 0.0)
        m = jnp.mean(h, axis=0, keepdims=True)
        v = jnp.mean((h - m) * (h - m), axis=0, keepdims=True)
        h = (h - m) * lax.rsqrt(v + 1e-5) * g1_ref[...] + be1_ref[...]

        h = jnp.maximum(
            jnp.dot(h, w2_ref[...], preferred_element_type=jnp.float32,
                    precision=hp) + b2_ref[...], 0.0)
        m = jnp.mean(h, axis=0, keepdims=True)
        v = jnp.mean((h - m) * (h - m), axis=0, keepdims=True)
        h = (h - m) * lax.rsqrt(v + 1e-5) * g2_ref[...] + be2_ref[...]

        out_ref[...] = jnp.dot(
            h, w3_ref[...], preferred_element_type=jnp.float32,
            precision=hp) + b3_ref[...]

    return pl.pallas_call(
        body,
        out_shape=jax.ShapeDtypeStruct((B, 1), jnp.float32),
    )(embt, xc, w1e, w1c, b1, g1, be1, w2, b2, g2, be2, w3, b3, gc, bc)


def kernel(x_cont, x_emb, tables, W1, b1, g1, be1, W2, b2, g2, be2, W3, b3,
           gc, bc):
    tbl = tables.transpose(0, 2, 1)
    idx3 = x_emb.T.reshape(F, 8, 512)
    tail3 = tbl[:, :, _VMAIN:].reshape(_NG, 8, _TAILW)
    embt = _sc_gather_t(tbl, idx3, tail3).reshape(F * D, B)
    out = _mlp(
        embt, x_cont,
        W1[:F * D], W1[F * D:],
        b1.reshape(1, -1), g1.reshape(1, -1), be1.reshape(1, -1),
        W2, b2.reshape(1, -1), g2.reshape(1, -1), be2.reshape(1, -1),
        W3, b3.reshape(1, -1), gc.reshape(1, -1), bc.reshape(1, -1),
    )
    return out


# speedup vs baseline: 1.0826x; 1.0826x over previous
import functools

import jax
import jax.numpy as jnp
from jax import lax
from jax.experimental import pallas as pl
from jax.experimental.pallas import tpu as pltpu
from jax.experimental.pallas import tpu_sc as plsc

B = 4096
F = 26
V = 100000
D = 16
C = 13

_NW = 32
_R = F * D
_RPW = _R // _NW


def _sc_gather_t(tbl_lin, idx_t, nrows):
    mesh = plsc.VectorSubcoreMesh(core_axis_name="c", subcore_axis_name="s")
    rpw = nrows // _NW

    @functools.partial(
        pl.kernel,
        mesh=mesh,
        out_type=jax.ShapeDtypeStruct((nrows, B), jnp.float32),
        scratch_types=[
            pltpu.VMEM((V,), jnp.float32),
            pltpu.VMEM((B,), jnp.int32),
            pltpu.VMEM((B,), jnp.float32),
        ],
        compiler_params=pltpu.CompilerParams(
            use_tc_tiling_on_sc=False, needs_layout_passes=False),
    )
    def k(tbl_hbm, idx_hbm, out_hbm, row_v, idx_v, out_v):
        wid = lax.axis_index("s") * 2 + lax.axis_index("c")

        def do_row(i, carry):
            r = wid * rpw + i
            f = r // D
            pltpu.sync_copy(tbl_hbm.at[r], row_v)
            pltpu.sync_copy(idx_hbm.at[f], idx_v)

            def extract(j, c2):
                vi = idx_v[pl.ds(j * 16, 16)]
                out_v[pl.ds(j * 16, 16)] = plsc.load_gather(row_v, [vi])
                return c2

            lax.fori_loop(0, B // 16, extract, 0)
            pltpu.sync_copy(out_v, out_hbm.at[r])
            return carry

        lax.fori_loop(0, rpw, do_row, 0)

    return k(tbl_lin, idx_t)


def _mlp(embta, embtb, xc, w1a, w1b, w1c, b1, g1, be1, w2, b2, g2, be2, w3,
         b3, gc, bc):
    def body(embta_ref, embtb_ref, xc_ref, w1a_ref, w1b_ref, w1c_ref,
             b1_ref, g1_ref, be1_ref, w2_ref, b2_ref, g2_ref, be2_ref,
             w3_ref, b3_ref, gc_ref, bc_ref, out_ref):
        hp = jax.lax.Precision.HIGHEST
        x = xc_ref[...]
        m = jnp.mean(x, axis=0, keepdims=True)
        v = jnp.mean((x - m) * (x - m), axis=0, keepdims=True)
        xn = (x - m) * lax.rsqrt(v + 1e-5) * gc_ref[...] + bc_ref[...]

        dn = (((0,), (0,)), ((), ()))
        h = jax.lax.dot_general(
            embta_ref[...], w1a_ref[...], dimension_numbers=dn,
            preferred_element_type=jnp.float32, precision=hp)
        h = h + jax.lax.dot_general(
            embtb_ref[...], w1b_ref[...], dimension_numbers=dn,
            preferred_element_type=jnp.float32, precision=hp)
        h = h + jnp.dot(xn, w1c_ref[...],
                        preferred_element_type=jnp.float32, precision=hp)
        h = jnp.maximum(h + b1_ref[...], 0.0)
        m = jnp.mean(h, axis=0, keepdims=True)
        v = jnp.mean((h - m) * (h - m), axis=0, keepdims=True)
        h = (h - m) * lax.rsqrt(v + 1e-5) * g1_ref[...] + be1_ref[...]

        h = jnp.maximum(
            jnp.dot(h, w2_ref[...], preferred_element_type=jnp.float32,
                    precision=hp) + b2_ref[...], 0.0)
        m = jnp.mean(h, axis=0, keepdims=True)
        v = jnp.mean((h - m) * (h - m), axis=0, keepdims=True)
        h = (h - m) * lax.rsqrt(v + 1e-5) * g2_ref[...] + be2_ref[...]

        out_ref[...] = jnp.dot(
            h, w3_ref[...], preferred_element_type=jnp.float32,
            precision=hp) + b3_ref[...]

    return pl.pallas_call(
        body,
        out_shape=jax.ShapeDtypeStruct((B, 1), jnp.float32),
    )(embta, embtb, xc, w1a, w1b, w1c, b1, g1, be1, w2, b2, g2, be2, w3,
      b3, gc, bc)


_FSPLIT = 16


def kernel(x_cont, x_emb, tables, W1, b1, g1, be1, W2, b2, g2, be2, W3, b3,
           gc, bc):
    idx_t = x_emb.T
    ra = _FSPLIT * D
    tbl_a = tables[:_FSPLIT].transpose(0, 2, 1).reshape(ra, V)
    tbl_b = tables[_FSPLIT:].transpose(0, 2, 1).reshape(_R - ra, V)
    embta = _sc_gather_t(tbl_a, idx_t[:_FSPLIT], ra)
    embtb = _sc_gather_t(tbl_b, idx_t[_FSPLIT:], _R - ra)
    out = _mlp(
        embta, embtb, x_cont,
        W1[:ra], W1[ra:_R], W1[_R:],
        b1.reshape(1, -1), g1.reshape(1, -1), be1.reshape(1, -1),
        W2, b2.reshape(1, -1), g2.reshape(1, -1), be2.reshape(1, -1),
        W3, b3.reshape(1, -1), gc.reshape(1, -1), bc.reshape(1, -1),
    )
    return out


# speedup vs baseline: 1.2824x; 1.1845x over previous
import functools

import jax
import jax.numpy as jnp
from jax import lax
from jax.experimental import pallas as pl
from jax.experimental.pallas import tpu as pltpu
from jax.experimental.pallas import tpu_sc as plsc

B = 4096
F = 26
V = 100000
D = 16
C = 13

_NW = 32
_R = F * D
_RPW = _R // _NW


def _sc_gather_t(tbl_lin, idx_t):
    mesh = plsc.VectorSubcoreMesh(core_axis_name="c", subcore_axis_name="s")

    @functools.partial(
        pl.kernel,
        mesh=mesh,
        out_type=jax.ShapeDtypeStruct((_R, B), jnp.float32),
        scratch_types=[
            pltpu.VMEM((V,), jnp.float32),
            pltpu.VMEM((B,), jnp.int32),
            pltpu.VMEM((B,), jnp.float32),
        ],
        compiler_params=pltpu.CompilerParams(
            use_tc_tiling_on_sc=False, needs_layout_passes=False),
    )
    def k(tbl_hbm, idx_hbm, out_hbm, row_v, idx_v, out_v):
        wid = lax.axis_index("s") * 2 + lax.axis_index("c")

        def do_row(i, carry):
            r = wid * _RPW + i
            f = r // D
            pltpu.sync_copy(tbl_hbm.at[r], row_v)
            pltpu.sync_copy(idx_hbm.at[f], idx_v)

            def extract(j, c2):
                vi = idx_v[pl.ds(j * 16, 16)]
                out_v[pl.ds(j * 16, 16)] = plsc.load_gather(row_v, [vi])
                return c2

            lax.fori_loop(0, B // 16, extract, 0)
            pltpu.sync_copy(out_v, out_hbm.at[r])
            return carry

        lax.fori_loop(0, _RPW, do_row, 0)

    return k(tbl_lin, idx_t)


def _mlp(embt, xc, w1e, w1c, b1, g1, be1, w2, b2, g2, be2, w3, b3, gc, bc):
    def body(embt_ref, xc_ref, w1e_ref, w1c_ref, b1_ref, g1_ref, be1_ref,
             w2_ref, b2_ref, g2_ref, be2_ref, w3_ref, b3_ref, gc_ref,
             bc_ref, out_ref):
        hp = jax.lax.Precision.HIGHEST
        x = xc_ref[...]
        m = jnp.mean(x, axis=0, keepdims=True)
        v = jnp.mean((x - m) * (x - m), axis=0, keepdims=True)
        xn = (x - m) * lax.rsqrt(v + 1e-5) * gc_ref[...] + bc_ref[...]

        h = jax.lax.dot_general(
            embt_ref[...], w1e_ref[...],
            dimension_numbers=(((0,), (0,)), ((), ())),
            preferred_element_type=jnp.float32, precision=hp)
        h = h + jnp.dot(xn, w1c_ref[...],
                        preferred_element_type=jnp.float32, precision=hp)
        h = jnp.maximum(h + b1_ref[...], 0.0)
        m = jnp.mean(h, axis=0, keepdims=True)
        v = jnp.mean((h - m) * (h - m), axis=0, keepdims=True)
        h = (h - m) * lax.rsqrt(v + 1e-5) * g1_ref[...] + be1_ref[...]

        h = jnp.maximum(
            jnp.dot(h, w2_ref[...], preferred_element_type=jnp.float32,
                    precision=hp) + b2_ref[...], 0.0)
        m = jnp.mean(h, axis=0, keepdims=True)
        v = jnp.mean((h - m) * (h - m), axis=0, keepdims=True)
        h = (h - m) * lax.rsqrt(v + 1e-5) * g2_ref[...] + be2_ref[...]

        out_ref[...] = jnp.dot(
            h, w3_ref[...], preferred_element_type=jnp.float32,
            precision=hp) + b3_ref[...]

    return pl.pallas_call(
        body,
        out_shape=jax.ShapeDtypeStruct((B, 1), jnp.float32),
    )(embt, xc, w1e, w1c, b1, g1, be1, w2, b2, g2, be2, w3, b3, gc, bc)


def kernel(x_cont, x_emb, tables, W1, b1, g1, be1, W2, b2, g2, be2, W3, b3,
           gc, bc):
    tbl_lin = tables.transpose(0, 2, 1).reshape(_R, V)
    embt = _sc_gather_t(tbl_lin, x_emb.T)
    out = _mlp(
        embt, x_cont,
        W1[:_R], W1[_R:],
        b1.reshape(1, -1), g1.reshape(1, -1), be1.reshape(1, -1),
        W2, b2.reshape(1, -1), g2.reshape(1, -1), be2.reshape(1, -1),
        W3, b3.reshape(1, -1), gc.reshape(1, -1), bc.reshape(1, -1),
    )
    return out
